# Initial kernel scaffold; baseline (speedup 1.0000x reference)
import jax
import jax.numpy as jnp
from jax.experimental import pallas as pl


def kernel(x_tokens, y_tokens, lane_indices, x_table, y_table, pos_table, lane_table):
    raise NotImplementedError("write your pallas kernel here")


# speedup vs baseline: 7.9793x; 7.9793x over previous
import functools

import jax
import jax.numpy as jnp
from jax import lax
from jax.experimental import pallas as pl
from jax.experimental.pallas import tpu as pltpu
from jax.experimental.pallas import tpu_sc as plsc

BATCH = 4096
SEQ = 200
DIM = 64
NBINS_X = 1000
NY = 201
NLANE = 8

NUM_CORES = 2
NUM_SUBCORES = 16
NUM_WORKERS = NUM_CORES * NUM_SUBCORES
ROWS_PER_W = BATCH // NUM_WORKERS
LANES = 16
DBLK = DIM // LANES
NGROUP = (SEQ + LANES - 1) // LANES
SEQ_PAD = NGROUP * LANES


def _body(
    xtok_hbm,
    ytok_hbm,
    lidx_hbm,
    xtab_hbm,
    ytab_hbm,
    pos_hbm,
    lane_hbm,
    out_hbm,
    xtab_v,
    ytab_v,
    pos_v,
    lane_v,
    lidx_v,
    xtok_v,
    ytok_v,
    out_v,
):
    wid = lax.axis_index("s") * NUM_CORES + lax.axis_index("c")
    base_b = wid * ROWS_PER_W

    zero16 = jnp.zeros((LANES,), jnp.int32)
    xtok_v[pl.ds(SEQ_PAD - LANES, LANES)] = zero16
    ytok_v[pl.ds(SEQ_PAD - LANES, LANES)] = zero16

    pltpu.sync_copy(xtab_hbm, xtab_v)
    pltpu.sync_copy(ytab_hbm, ytab_v)
    pltpu.sync_copy(pos_hbm, pos_v.at[pl.ds(0, SEQ * DIM)])
    pltpu.sync_copy(lane_hbm, lane_v)
    pltpu.sync_copy(lidx_hbm.at[pl.ds(base_b, ROWS_PER_W)], lidx_v.at[pl.ds(0, ROWS_PER_W)])

    for r in range(NLANE):
        for k in range(DBLK):
            sl = pl.ds(r * DIM + LANES * k, LANES)
            lane_v[sl] = lane_v[sl] * 10.0

    def row_body(b, carry):
        bb = base_b + b
        pltpu.sync_copy(xtok_hbm.at[pl.ds(bb * SEQ, SEQ)], xtok_v.at[pl.ds(0, SEQ)])
        pltpu.sync_copy(ytok_hbm.at[pl.ds(bb * SEQ, SEQ)], ytok_v.at[pl.ds(0, SEQ)])
        l = lidx_v[pl.ds(b, LANES)][0]
        lvec = [lane_v[pl.ds(l * DIM + LANES * k, LANES)] for k in range(DBLK)]

        def grp_body(g, c2):
            s0 = g * LANES
            txv = xtok_v[pl.ds(s0, LANES)]
            tyv = ytok_v[pl.ds(s0, LANES)]
            for j in range(LANES):
                tx = txv[j] * DIM
                ty = tyv[j] * DIM
                s = s0 + j
                for k in range(DBLK):
                    off = LANES * k
                    out_v[s, pl.ds(off, LANES)] = (
                        xtab_v[pl.ds(tx + off, LANES)]
                        + ytab_v[pl.ds(ty + off, LANES)]
                        + pos_v[pl.ds(s * DIM + off, LANES)]
                        + lvec[k]
                    )
            return c2

        lax.fori_loop(0, NGROUP, grp_body, 0)
        pltpu.sync_copy(out_v.at[pl.ds(0, SEQ)], out_hbm.at[bb])
        return carry

    lax.fori_loop(0, ROWS_PER_W, row_body, 0)


@jax.jit
def _run(xtok, ytok, lidx, xtab, ytab, pos, lane):
    mesh = plsc.VectorSubcoreMesh(core_axis_name="c", subcore_axis_name="s")
    return pl.kernel(
        _body,
        out_type=jax.ShapeDtypeStruct((BATCH, SEQ, DIM), jnp.float32),
        mesh=mesh,
        scratch_types=[
            pltpu.VMEM((NBINS_X * DIM,), jnp.float32),
            pltpu.VMEM((NY * DIM,), jnp.float32),
            pltpu.VMEM((SEQ_PAD * DIM,), jnp.float32),
            pltpu.VMEM((NLANE * DIM,), jnp.float32),
            pltpu.VMEM((ROWS_PER_W + LANES,), jnp.int32),
            pltpu.VMEM((SEQ_PAD,), jnp.int32),
            pltpu.VMEM((SEQ_PAD,), jnp.int32),
            pltpu.VMEM((SEQ_PAD, DIM), jnp.float32),
        ],
    )(xtok, ytok, lidx, xtab, ytab, pos, lane)


def kernel(x_tokens, y_tokens, lane_indices, x_table, y_table, pos_table, lane_table):
    return _run(
        x_tokens.astype(jnp.int32).reshape(-1),
        y_tokens.astype(jnp.int32).reshape(-1),
        lane_indices.astype(jnp.int32),
        x_table.reshape(-1),
        y_table.reshape(-1),
        pos_table.reshape(-1),
        lane_table.reshape(-1),
    )


# speedup vs baseline: 10.1308x; 1.2696x over previous
import functools

import jax
import jax.numpy as jnp
from jax import lax
from jax.experimental import pallas as pl
from jax.experimental.pallas import tpu as pltpu
from jax.experimental.pallas import tpu_sc as plsc

BATCH = 4096
SEQ = 200
DIM = 64
NBINS_X = 1000
NY = 201
NLANE = 8

NUM_CORES = 2
NUM_SUBCORES = 16
NUM_WORKERS = NUM_CORES * NUM_SUBCORES
ROWS_PER_W = BATCH // NUM_WORKERS
LANES = 16
DBLK = DIM // LANES
NGROUP = (SEQ + LANES - 1) // LANES
SEQ_PAD = NGROUP * LANES
ROW_F = SEQ * DIM
ROWBUF = SEQ_PAD * DIM
TOKBUF = SEQ_PAD + 8


def _body(
    xtok_hbm,
    ytok_hbm,
    lidx_hbm,
    xtab_hbm,
    ytab_hbm,
    pos_hbm,
    lane_hbm,
    out_hbm,
    xtab_v,
    ytab_v,
    pos_v,
    lane_v,
    lidx_v,
    tokx_v,
    toky_v,
    out_v,
    sem_o,
    sem_t,
):
    wid = lax.axis_index("s") * NUM_CORES + lax.axis_index("c")
    base_b = wid * ROWS_PER_W

    zero16 = jnp.zeros((LANES,), jnp.int32)
    for t in range(2):
        tokx_v[pl.ds(t * TOKBUF + SEQ, LANES)] = zero16
        toky_v[pl.ds(t * TOKBUF + SEQ, LANES)] = zero16

    pltpu.sync_copy(xtab_hbm, xtab_v)
    pltpu.sync_copy(ytab_hbm, ytab_v)
    pltpu.sync_copy(pos_hbm, pos_v.at[pl.ds(0, SEQ * DIM)])
    pltpu.sync_copy(lane_hbm, lane_v)
    pltpu.sync_copy(
        lidx_hbm.at[pl.ds(base_b, ROWS_PER_W)], lidx_v.at[pl.ds(0, ROWS_PER_W)]
    )

    for r in range(NLANE):
        for k in range(DBLK):
            sl = pl.ds(r * DIM + LANES * k, LANES)
            lane_v[sl] = lane_v[sl] * 10.0

    def tok_copies(b, tbuf):
        src = pl.ds((base_b + b) * SEQ, SEQ)
        dst = pl.ds(tbuf * TOKBUF, SEQ)
        return (
            pltpu.make_async_copy(xtok_hbm.at[src], tokx_v.at[dst], sem_t),
            pltpu.make_async_copy(ytok_hbm.at[src], toky_v.at[dst], sem_t),
        )

    def out_copy(bb, tbuf):
        return pltpu.make_async_copy(
            out_v.at[pl.ds(tbuf * ROWBUF, ROW_F)],
            out_hbm.at[pl.ds(bb * ROW_F, ROW_F)],
            sem_o.at[tbuf],
        )

    for cp in tok_copies(0, 0):
        cp.start()

    def row_body(b, carry):
        t = lax.bitwise_and(b, 1)
        bb = base_b + b

        for cp in tok_copies(b, t):
            cp.wait()

        @pl.when(b < ROWS_PER_W - 1)
        def _():
            for cp in tok_copies(b + 1, 1 - t):
                cp.start()

        @pl.when(b >= 2)
        def _():
            out_copy(bb, t).wait()

        l = lidx_v[pl.ds(b, LANES)][0]
        lvec = [lane_v[pl.ds(l * DIM + LANES * k, LANES)] for k in range(DBLK)]
        obase = t * ROWBUF

        @plsc.parallel_loop(0, NGROUP, unroll=2)
        def grp(g):
            s0 = g * LANES
            txv = tokx_v[pl.ds(t * TOKBUF + s0, LANES)]
            tyv = toky_v[pl.ds(t * TOKBUF + s0, LANES)]
            for j in range(LANES):
                tx = txv[j] * DIM
                ty = tyv[j] * DIM
                so = (s0 + j) * DIM
                for k in range(DBLK):
                    off = LANES * k
                    out_v[pl.ds(obase + so + off, LANES)] = (
                        xtab_v[pl.ds(tx + off, LANES)] + ytab_v[pl.ds(ty + off, LANES)]
                    ) + (pos_v[pl.ds(so + off, LANES)] + lvec[k])

        out_copy(bb, t).start()
        return carry

    lax.fori_loop(0, ROWS_PER_W, row_body, 0)

    def drain(q, carry):
        out_copy(base_b + ROWS_PER_W - 2 + q, lax.bitwise_and(q, 1)).wait()
        return carry

    lax.fori_loop(0, 2, drain, 0)


@jax.jit
def _run(xtok, ytok, lidx, xtab, ytab, pos, lane):
    mesh = plsc.VectorSubcoreMesh(core_axis_name="c", subcore_axis_name="s")
    flat = pl.kernel(
        _body,
        out_type=jax.ShapeDtypeStruct((BATCH * SEQ * DIM,), jnp.float32),
        mesh=mesh,
        scratch_types=[
            pltpu.VMEM((NBINS_X * DIM,), jnp.float32),
            pltpu.VMEM((NY * DIM,), jnp.float32),
            pltpu.VMEM((SEQ_PAD * DIM,), jnp.float32),
            pltpu.VMEM((NLANE * DIM,), jnp.float32),
            pltpu.VMEM((ROWS_PER_W + LANES,), jnp.int32),
            pltpu.VMEM((2 * TOKBUF,), jnp.int32),
            pltpu.VMEM((2 * TOKBUF,), jnp.int32),
            pltpu.VMEM((2 * ROWBUF,), jnp.float32),
            pltpu.SemaphoreType.DMA((2,)),
            pltpu.SemaphoreType.DMA,
        ],
    )(xtok, ytok, lidx, xtab, ytab, pos, lane)
    return flat.reshape(BATCH, SEQ, DIM)


def kernel(x_tokens, y_tokens, lane_indices, x_table, y_table, pos_table, lane_table):
    return _run(
        x_tokens.astype(jnp.int32).reshape(-1),
        y_tokens.astype(jnp.int32).reshape(-1),
        lane_indices.astype(jnp.int32),
        x_table.reshape(-1),
        y_table.reshape(-1),
        pos_table.reshape(-1),
        lane_table.reshape(-1),
    )


# speedup vs baseline: 14.6881x; 1.4498x over previous
import functools

import jax
import jax.numpy as jnp
from jax import lax
from jax.experimental import pallas as pl
from jax.experimental.pallas import tpu as pltpu
from jax.experimental.pallas import tpu_sc as plsc

BATCH = 4096
SEQ = 200
DIM = 64
NBINS_X = 1000
NY = 201
NLANE = 8

NUM_CORES = 2
NUM_SUBCORES = 16
NUM_WORKERS = NUM_CORES * NUM_SUBCORES
ROWS_PER_W = BATCH // NUM_WORKERS
LANES = 16
DBLK = DIM // LANES
NGROUP = (SEQ + LANES - 1) // LANES
SEQ_PAD = NGROUP * LANES
ROW_F = SEQ * DIM
ROWBUF = SEQ_PAD * DIM
TOKBUF = SEQ_PAD + 8


def _body(
    xtok_hbm,
    ytok_hbm,
    lidx_hbm,
    xtab_hbm,
    ytab_hbm,
    pos_hbm,
    lane_hbm,
    out_hbm,
    xtab_v,
    ytab_v,
    pos_v,
    lane_v,
    lidx_v,
    tokx_v,
    toky_v,
    out_v,
    sem_o,
    sem_t,
):
    wid = lax.axis_index("s") * NUM_CORES + lax.axis_index("c")
    base_b = wid * ROWS_PER_W

    zero16 = jnp.zeros((LANES,), jnp.int32)
    for t in range(2):
        tokx_v[pl.ds(t * TOKBUF + SEQ, LANES)] = zero16
        toky_v[pl.ds(t * TOKBUF + SEQ, LANES)] = zero16

    pltpu.sync_copy(xtab_hbm, xtab_v)
    pltpu.sync_copy(ytab_hbm, ytab_v)
    pltpu.sync_copy(pos_hbm, pos_v.at[pl.ds(0, SEQ * DIM)])
    pltpu.sync_copy(lane_hbm, lane_v)
    pltpu.sync_copy(
        lidx_hbm.at[pl.ds(base_b, ROWS_PER_W)], lidx_v.at[pl.ds(0, ROWS_PER_W)]
    )

    for r in range(NLANE):
        for k in range(DBLK):
            sl = pl.ds(r * DIM + LANES * k, LANES)
            lane_v[sl] = lane_v[sl] * 10.0

    def tok_copies(b, tbuf):
        src = pl.ds((base_b + b) * SEQ, SEQ)
        dst = pl.ds(tbuf * TOKBUF, SEQ)
        return (
            pltpu.make_async_copy(xtok_hbm.at[src], tokx_v.at[dst], sem_t),
            pltpu.make_async_copy(ytok_hbm.at[src], toky_v.at[dst], sem_t),
        )

    def out_copy(bb, tbuf):
        return pltpu.make_async_copy(
            out_v.at[pl.ds(tbuf * ROWBUF, ROW_F)],
            out_hbm.at[pl.ds(bb * ROW_F, ROW_F)],
            sem_o.at[tbuf],
        )

    for cp in tok_copies(0, 0):
        cp.start()

    def row_body(b, carry):
        t = lax.bitwise_and(b, 1)
        bb = base_b + b

        for cp in tok_copies(b, t):
            cp.wait()

        @pl.when(b < ROWS_PER_W - 1)
        def _():
            for cp in tok_copies(b + 1, 1 - t):
                cp.start()

        @pl.when(b >= 2)
        def _():
            out_copy(bb, t).wait()

        l = lidx_v[pl.ds(b, LANES)][0]
        lvec = [lane_v[pl.ds(l * DIM + LANES * k, LANES)] for k in range(DBLK)]
        obase = t * ROWBUF

        @plsc.parallel_loop(0, NGROUP, unroll=2)
        def grp(g):
            s0 = g * LANES
            txv = tokx_v[pl.ds(t * TOKBUF + s0, LANES)]
            tyv = toky_v[pl.ds(t * TOKBUF + s0, LANES)]
            for j in range(0, LANES, 2):
                tx0 = txv[j] * DIM
                ty0 = tyv[j] * DIM
                tx1 = txv[j + 1] * DIM
                ty1 = tyv[j + 1] * DIM
                so0 = (s0 + j) * DIM
                so1 = so0 + DIM
                x0 = [xtab_v[pl.ds(tx0 + LANES * k, LANES)] for k in range(DBLK)]
                y0 = [ytab_v[pl.ds(ty0 + LANES * k, LANES)] for k in range(DBLK)]
                p0 = [pos_v[pl.ds(so0 + LANES * k, LANES)] for k in range(DBLK)]
                x1 = [xtab_v[pl.ds(tx1 + LANES * k, LANES)] for k in range(DBLK)]
                y1 = [ytab_v[pl.ds(ty1 + LANES * k, LANES)] for k in range(DBLK)]
                p1 = [pos_v[pl.ds(so1 + LANES * k, LANES)] for k in range(DBLK)]
                for k in range(DBLK):
                    off = LANES * k
                    out_v[pl.ds(obase + so0 + off, LANES)] = (x0[k] + y0[k]) + (
                        p0[k] + lvec[k]
                    )
                for k in range(DBLK):
                    off = LANES * k
                    out_v[pl.ds(obase + so1 + off, LANES)] = (x1[k] + y1[k]) + (
                        p1[k] + lvec[k]
                    )

        out_copy(bb, t).start()
        return carry

    lax.fori_loop(0, ROWS_PER_W, row_body, 0)

    def drain(q, carry):
        out_copy(base_b + ROWS_PER_W - 2 + q, lax.bitwise_and(q, 1)).wait()
        return carry

    lax.fori_loop(0, 2, drain, 0)


@jax.jit
def _run(xtok, ytok, lidx, xtab, ytab, pos, lane):
    mesh = plsc.VectorSubcoreMesh(core_axis_name="c", subcore_axis_name="s")
    flat = pl.kernel(
        _body,
        out_type=jax.ShapeDtypeStruct((BATCH * SEQ * DIM,), jnp.float32),
        mesh=mesh,
        scratch_types=[
            pltpu.VMEM((NBINS_X * DIM,), jnp.float32),
            pltpu.VMEM((NY * DIM,), jnp.float32),
            pltpu.VMEM((SEQ_PAD * DIM,), jnp.float32),
            pltpu.VMEM((NLANE * DIM,), jnp.float32),
            pltpu.VMEM((ROWS_PER_W + LANES,), jnp.int32),
            pltpu.VMEM((2 * TOKBUF,), jnp.int32),
            pltpu.VMEM((2 * TOKBUF,), jnp.int32),
            pltpu.VMEM((2 * ROWBUF,), jnp.float32),
            pltpu.SemaphoreType.DMA((2,)),
            pltpu.SemaphoreType.DMA,
        ],
    )(xtok, ytok, lidx, xtab, ytab, pos, lane)
    return flat.reshape(BATCH, SEQ, DIM)


def kernel(x_tokens, y_tokens, lane_indices, x_table, y_table, pos_table, lane_table):
    return _run(
        x_tokens.astype(jnp.int32).reshape(-1),
        y_tokens.astype(jnp.int32).reshape(-1),
        lane_indices.astype(jnp.int32),
        x_table.reshape(-1),
        y_table.reshape(-1),
        pos_table.reshape(-1),
        lane_table.reshape(-1),
    )


# speedup vs baseline: 19.8208x; 1.3494x over previous
import functools

import jax
import jax.numpy as jnp
from jax import lax
from jax.experimental import pallas as pl
from jax.experimental.pallas import tpu as pltpu
from jax.experimental.pallas import tpu_sc as plsc

BATCH = 4096
SEQ = 200
DIM = 64
NBINS_X = 1000
NY = 201
NLANE = 8

NUM_CORES = 2
NUM_SUBCORES = 16
NUM_WORKERS = NUM_CORES * NUM_SUBCORES
ROWS_PER_W = BATCH // NUM_WORKERS
LANES = 16
DBLK = DIM // LANES
NGROUP = (SEQ + LANES - 1) // LANES
SEQ_PAD = NGROUP * LANES
DPAD = 128
SPLIT = 96
GSPLIT = SPLIT // LANES
HB_ROWS = SEQ - SPLIT
HB_PAD = SEQ_PAD - SPLIT
TOKBUF = 2 * SEQ + LANES


def _body(
    xtok_hbm,
    ytok_hbm,
    lidx_hbm,
    xtab_hbm,
    ytab_hbm,
    pos_hbm,
    lane_hbm,
    out_hbm,
    xtab_v,
    ytab_v,
    pos_v,
    lane_v,
    lidx_v,
    tokx_v,
    toky_v,
    outa_v,
    outb_v,
    sem_a,
    sem_b,
    sem_t,
):
    wid = lax.axis_index("s") * NUM_CORES + lax.axis_index("c")
    base_b = wid * ROWS_PER_W

    zero16 = jnp.zeros((LANES,), jnp.int32)
    for t in range(2):
        tokx_v[pl.ds(t * TOKBUF + 2 * SEQ, LANES)] = zero16
        toky_v[pl.ds(t * TOKBUF + 2 * SEQ, LANES)] = zero16

    pltpu.sync_copy(xtab_hbm, xtab_v)
    pltpu.sync_copy(ytab_hbm, ytab_v)
    pltpu.sync_copy(pos_hbm, pos_v.at[pl.ds(0, SEQ * DIM)])
    pltpu.sync_copy(lane_hbm, lane_v)
    pltpu.sync_copy(
        lidx_hbm.at[pl.ds(base_b, ROWS_PER_W)], lidx_v.at[pl.ds(0, ROWS_PER_W)]
    )

    for r in range(NLANE):
        for k in range(DBLK):
            sl = pl.ds(r * DIM + LANES * k, LANES)
            lane_v[sl] = lane_v[sl] * 10.0

    def tok_copies(pair, tbuf):
        src = pl.ds((base_b + 2 * pair) * SEQ, 2 * SEQ)
        dst = pl.ds(tbuf * TOKBUF, 2 * SEQ)
        return (
            pltpu.make_async_copy(xtok_hbm.at[src], tokx_v.at[dst], sem_t),
            pltpu.make_async_copy(ytok_hbm.at[src], toky_v.at[dst], sem_t),
        )

    def copy_a(bb):
        return pltpu.make_async_copy(
            outa_v, out_hbm.at[bb, pl.ds(0, SPLIT)], sem_a
        )

    def copy_b(bb):
        return pltpu.make_async_copy(
            outb_v.at[pl.ds(0, HB_ROWS)], out_hbm.at[bb, pl.ds(SPLIT, HB_ROWS)], sem_b
        )

    for cp in tok_copies(0, 0):
        cp.start()

    NPAIRS = ROWS_PER_W // 2

    def compute_half(t, row_off, out_ref, b_local, g_lo, g_hi, s_base):
        l = lidx_v[pl.ds(b_local, LANES)][0]
        lvec = [lane_v[pl.ds(l * DIM + LANES * k, LANES)] for k in range(DBLK)]

        @plsc.parallel_loop(g_lo, g_hi, unroll=2)
        def grp(g):
            s0 = g * LANES
            r0 = s0 - s_base
            txv = tokx_v[pl.ds(t * TOKBUF + row_off + s0, LANES)]
            tyv = toky_v[pl.ds(t * TOKBUF + row_off + s0, LANES)]
            for j in range(0, LANES, 2):
                tx0 = txv[j] * DIM
                ty0 = tyv[j] * DIM
                tx1 = txv[j + 1] * DIM
                ty1 = tyv[j + 1] * DIM
                so0 = (s0 + j) * DIM
                so1 = so0 + DIM
                x0 = [xtab_v[pl.ds(tx0 + LANES * k, LANES)] for k in range(DBLK)]
                y0 = [ytab_v[pl.ds(ty0 + LANES * k, LANES)] for k in range(DBLK)]
                p0 = [pos_v[pl.ds(so0 + LANES * k, LANES)] for k in range(DBLK)]
                x1 = [xtab_v[pl.ds(tx1 + LANES * k, LANES)] for k in range(DBLK)]
                y1 = [ytab_v[pl.ds(ty1 + LANES * k, LANES)] for k in range(DBLK)]
                p1 = [pos_v[pl.ds(so1 + LANES * k, LANES)] for k in range(DBLK)]
                for k in range(DBLK):
                    off = LANES * k
                    out_ref[r0 + j, pl.ds(off, LANES)] = (x0[k] + y0[k]) + (
                        p0[k] + lvec[k]
                    )
                for k in range(DBLK):
                    off = LANES * k
                    out_ref[r0 + j + 1, pl.ds(off, LANES)] = (x1[k] + y1[k]) + (
                        p1[k] + lvec[k]
                    )

    def do_half_a(b, t, row_off, bb):
        @pl.when(b >= 1)
        def _():
            copy_a(bb).wait()

        compute_half(t, row_off, outa_v, b, 0, GSPLIT, 0)
        copy_a(bb).start()

    def do_half_b(b, t, row_off, bb):
        @pl.when(b >= 1)
        def _():
            copy_b(bb).wait()

        compute_half(t, row_off, outb_v, b, GSPLIT, NGROUP, SPLIT)
        copy_b(bb).start()

    def pair_body(p, carry):
        t = lax.bitwise_and(p, 1)
        bb0 = base_b + 2 * p

        for cp in tok_copies(p, t):
            cp.wait()

        @pl.when(p < NPAIRS - 1)
        def _():
            for cp in tok_copies(p + 1, 1 - t):
                cp.start()

        do_half_a(2 * p, t, 0, bb0)
        do_half_b(2 * p, t, 0, bb0)
        do_half_a(2 * p + 1, t, SEQ, bb0 + 1)
        do_half_b(2 * p + 1, t, SEQ, bb0 + 1)
        return carry

    lax.fori_loop(0, NPAIRS, pair_body, 0)

    copy_a(base_b + ROWS_PER_W - 1).wait()
    copy_b(base_b + ROWS_PER_W - 1).wait()


@jax.jit
def _run(xtok, ytok, lidx, xtab, ytab, pos, lane):
    mesh = plsc.VectorSubcoreMesh(core_axis_name="c", subcore_axis_name="s")
    flat = pl.kernel(
        _body,
        out_type=jax.ShapeDtypeStruct((BATCH, SEQ, DPAD), jnp.float32),
        mesh=mesh,
        scratch_types=[
            pltpu.VMEM((NBINS_X * DIM,), jnp.float32),
            pltpu.VMEM((NY * DIM,), jnp.float32),
            pltpu.VMEM((SEQ_PAD * DIM,), jnp.float32),
            pltpu.VMEM((NLANE * DIM,), jnp.float32),
            pltpu.VMEM((ROWS_PER_W + LANES,), jnp.int32),
            pltpu.VMEM((2 * TOKBUF,), jnp.int32),
            pltpu.VMEM((2 * TOKBUF,), jnp.int32),
            pltpu.VMEM((SPLIT, DPAD), jnp.float32),
            pltpu.VMEM((HB_PAD, DPAD), jnp.float32),
            pltpu.SemaphoreType.DMA,
            pltpu.SemaphoreType.DMA,
            pltpu.SemaphoreType.DMA,
        ],
    )(xtok, ytok, lidx, xtab, ytab, pos, lane)
    return flat[:, :, :DIM]


def kernel(x_tokens, y_tokens, lane_indices, x_table, y_table, pos_table, lane_table):
    return _run(
        x_tokens.astype(jnp.int32).reshape(-1),
        y_tokens.astype(jnp.int32).reshape(-1),
        lane_indices.astype(jnp.int32),
        x_table.reshape(-1),
        y_table.reshape(-1),
        pos_table.reshape(-1),
        lane_table.reshape(-1),
    )
